# Initial kernel scaffold; baseline (speedup 1.0000x reference)
import jax
import jax.numpy as jnp
from jax.experimental import pallas as pl


def kernel(nodes, table, r_table, t_table, hist_ua, hist_r, hist_t, W_agg, b_agg, W1, b1):
    raise NotImplementedError("write your pallas kernel here")


# speedup vs baseline: 4.3874x; 4.3874x over previous
import functools

import jax
import jax.numpy as jnp
from jax import lax
from jax.experimental import pallas as pl
from jax.experimental.pallas import tpu as pltpu
from jax.experimental.pallas import tpu_sc as plsc

B = 4096
L = 32
D = 64
N_R = 6
N_T = 64
NC = 2
NS = 16
NW = NC * NS
B_PER = B // NW
RPC = 128
NCHUNK = B_PER * L // RPC
G = 4
NGROUP = NCHUNK // G


def _sc_gather(nodes, hist_ua, hist_r, hist_t, table):
    mesh = plsc.VectorSubcoreMesh(core_axis_name="c", subcore_axis_name="s")

    @functools.partial(
        pl.kernel,
        out_type=(
            jax.ShapeDtypeStruct((B * L, D), jnp.float32),
            jax.ShapeDtypeStruct((B, L), jnp.int32),
            jax.ShapeDtypeStruct((B, L), jnp.int32),
            jax.ShapeDtypeStruct((B, D), jnp.float32),
        ),
        mesh=mesh,
        compiler_params=pltpu.CompilerParams(use_tc_tiling_on_sc=False),
        scratch_types=[
            pltpu.VMEM((B_PER,), jnp.int32),
            pltpu.VMEM((B_PER, L), jnp.int32),
            pltpu.VMEM((B_PER * L // RPC, RPC), jnp.int32),
            pltpu.VMEM((B_PER, L), jnp.int32),
            pltpu.VMEM((B_PER, L), jnp.int32),
            pltpu.VMEM((B_PER, D), jnp.float32),
            pltpu.VMEM((G, RPC, D), jnp.float32),
            pltpu.SemaphoreType.DMA,
            pltpu.SemaphoreType.DMA,
        ],
    )
    def k(nodes_hbm, ua_hbm, r_hbm, t_hbm, table_hbm,
          e_out, r_out, t_out, self_out,
          nodes_v, ids_v, idsw_v, rids_v, tids_v, self_v, rows_v, gsem, wsem):
        wid = lax.axis_index("s") * NC + lax.axis_index("c")
        base = wid * B_PER
        pltpu.sync_copy(nodes_hbm.at[pl.ds(base, B_PER)], nodes_v)
        c1 = pltpu.async_copy(ua_hbm.at[nodes_v], ids_v, gsem)
        c2 = pltpu.async_copy(r_hbm.at[nodes_v], rids_v, gsem)
        c3 = pltpu.async_copy(t_hbm.at[nodes_v], tids_v, gsem)
        c4 = pltpu.async_copy(table_hbm.at[nodes_v], self_v, gsem)
        c1.wait()
        c2.wait()
        c3.wait()
        c4.wait()
        w1 = pltpu.async_copy(rids_v, r_out.at[pl.ds(base, B_PER)], wsem)
        w2 = pltpu.async_copy(tids_v, t_out.at[pl.ds(base, B_PER)], wsem)
        w3 = pltpu.async_copy(self_v, self_out.at[pl.ds(base, B_PER)], wsem)
        for f in range(B_PER * L // 16):
            v = ids_v[(16 * f) // L, pl.ds((16 * f) % L, 16)]
            idsw_v[(16 * f) // RPC, pl.ds((16 * f) % RPC, 16)] = v
        ebase = base * L

        def group(g, _):
            hs = []
            for i in range(G):
                r = g * G + i
                hs.append(pltpu.async_copy(
                    table_hbm.at[idsw_v.at[r]], rows_v.at[i], gsem))
            for i in range(G):
                r = g * G + i
                hs[i].wait()
                pltpu.async_copy(
                    rows_v.at[i], e_out.at[pl.ds(ebase + r * RPC, RPC)],
                    wsem).wait()
            return 0

        lax.fori_loop(0, NGROUP, group, 0)
        w1.wait()
        w2.wait()
        w3.wait()

    return k(nodes, hist_ua, hist_r, hist_t, table)


BB = 128


def _tc_dense(e_n, rids, tids, self_f, r_table, t_table, W_agg, b_agg, W1, b1):
    def body(e_ref, rid_ref, tid_ref, self_ref, rt_ref, tt_ref, wa_ref,
             ba_ref, w1_ref, b1_ref, o_ref):
        wn = wa_ref[0:D, :]
        rw = jnp.dot(rt_ref[...], wa_ref[D:2 * D, :],
                     preferred_element_type=jnp.float32)
        tw = jnp.dot(tt_ref[...], wa_ref[2 * D:3 * D, :],
                     preferred_element_type=jnp.float32)
        x = e_ref[...].reshape(BB * L, D)
        acc = jnp.dot(x, wn, preferred_element_type=jnp.float32)
        rid3 = lax.broadcast_in_dim(rid_ref[...], (BB, L, N_R), (0, 1))
        ohr = (rid3 == lax.broadcasted_iota(jnp.int32, (BB, L, N_R), 2)
               ).astype(jnp.float32).reshape(BB * L, N_R)
        acc = acc + jnp.dot(ohr, rw, preferred_element_type=jnp.float32)
        tid3 = lax.broadcast_in_dim(tid_ref[...], (BB, L, N_T), (0, 1))
        oht = (tid3 == lax.broadcasted_iota(jnp.int32, (BB, L, N_T), 2)
               ).astype(jnp.float32).reshape(BB * L, N_T)
        acc = acc + jnp.dot(oht, tw, preferred_element_type=jnp.float32)
        h = jnp.maximum(acc + ba_ref[...], 0.0)
        neigh = jnp.mean(h.reshape(BB, L, D), axis=1)
        comb = jnp.concatenate([self_ref[...], neigh], axis=1)
        o_ref[...] = jnp.maximum(
            jnp.dot(comb, w1_ref[...], preferred_element_type=jnp.float32)
            + b1_ref[...], 0.0)

    grid = (B // BB,)
    return pl.pallas_call(
        body,
        grid=grid,
        in_specs=[
            pl.BlockSpec((BB, L, D), lambda i: (i, 0, 0)),
            pl.BlockSpec((BB, L), lambda i: (i, 0)),
            pl.BlockSpec((BB, L), lambda i: (i, 0)),
            pl.BlockSpec((BB, D), lambda i: (i, 0)),
            pl.BlockSpec((N_R, D), lambda i: (0, 0)),
            pl.BlockSpec((N_T, D), lambda i: (0, 0)),
            pl.BlockSpec((3 * D, D), lambda i: (0, 0)),
            pl.BlockSpec((1, D), lambda i: (0, 0)),
            pl.BlockSpec((2 * D, D), lambda i: (0, 0)),
            pl.BlockSpec((1, D), lambda i: (0, 0)),
        ],
        out_specs=pl.BlockSpec((BB, D), lambda i: (i, 0)),
        out_shape=jax.ShapeDtypeStruct((B, D), jnp.float32),
    )(e_n, rids, tids, self_f, r_table, t_table, W_agg,
      b_agg.reshape(1, D), W1, b1.reshape(1, D))


def kernel(nodes, table, r_table, t_table, hist_ua, hist_r, hist_t,
           W_agg, b_agg, W1, b1):
    nodes = nodes.astype(jnp.int32)
    hist_ua = hist_ua.astype(jnp.int32)
    hist_r = hist_r.astype(jnp.int32)
    hist_t = hist_t.astype(jnp.int32)
    e_n, rids, tids, self_f = _sc_gather(nodes, hist_ua, hist_r, hist_t, table)
    return _tc_dense(e_n.reshape(B, L, D), rids, tids, self_f,
                     r_table, t_table, W_agg, b_agg, W1, b1)
